# baseline (device time: 8787 ns/iter reference)
import jax
import jax.numpy as jnp
from jax import lax
from jax.experimental import pallas as pl
from jax.experimental.pallas import tpu as pltpu

_NCHUNK = 8


def kernel(x):
    m_per, n_per = x.shape
    bm = m_per // _NCHUNK
    rows = bm // 128

    def body(x_ref, out_ref, partial_ref, peer_ref, send_sems, recv_sems):
        my_x = lax.axis_index("x")
        my_y = lax.axis_index("y")
        peer = (my_x, 1 - my_y)
        barrier_sem = pltpu.get_barrier_semaphore()

        def chunk_rdma(k):
            return pltpu.make_async_remote_copy(
                src_ref=partial_ref.at[pl.ds(k * rows, rows)],
                dst_ref=peer_ref.at[pl.ds(k * rows, rows)],
                send_sem=send_sems.at[k],
                recv_sem=recv_sems.at[k],
                device_id=peer,
                device_id_type=pl.DeviceIdType.MESH,
            )

        pl.semaphore_signal(
            barrier_sem, inc=1, device_id=peer,
            device_id_type=pl.DeviceIdType.MESH,
        )

        for k in range(_NCHUNK):
            s = jnp.sum(
                x_ref[pl.ds(k * bm, bm), :].astype(jnp.float32), axis=1
            )
            partial_ref[pl.ds(k * rows, rows), :] = s.reshape(rows, 128)
            if k == 0:
                pl.semaphore_wait(barrier_sem, 1)
            chunk_rdma(k).start()

        for k in range(_NCHUNK):
            rdma = chunk_rdma(k)
            rdma.wait_send()
            rdma.wait_recv()
        out_ref[:, :] = partial_ref[:, :] + peer_ref[:, :]

    out = pl.pallas_call(
        body,
        out_shape=jax.ShapeDtypeStruct((m_per // 128, 128), jnp.float32),
        in_specs=[pl.BlockSpec(memory_space=pltpu.VMEM)],
        out_specs=pl.BlockSpec(memory_space=pltpu.VMEM),
        scratch_shapes=[
            pltpu.VMEM((m_per // 128, 128), jnp.float32),
            pltpu.VMEM((m_per // 128, 128), jnp.float32),
            pltpu.SemaphoreType.DMA((_NCHUNK,)),
            pltpu.SemaphoreType.DMA((_NCHUNK,)),
        ],
        compiler_params=pltpu.CompilerParams(collective_id=0),
    )(x)
    return out.reshape(m_per, 1)


# device time: 8423 ns/iter; 1.0432x vs baseline; 1.0432x over previous
import jax
import jax.numpy as jnp
from jax import lax
from jax.experimental import pallas as pl
from jax.experimental.pallas import tpu as pltpu

_NCHUNK = 4


def kernel(x):
    m_per, n_per = x.shape
    bm = m_per // _NCHUNK
    rows = bm // 128

    def body(x_ref, out_ref, partial_ref, peer_ref, send_sems, recv_sems):
        my_x = lax.axis_index("x")
        my_y = lax.axis_index("y")
        peer = (my_x, 1 - my_y)
        barrier_sem = pltpu.get_barrier_semaphore()

        def chunk_rdma(k):
            return pltpu.make_async_remote_copy(
                src_ref=partial_ref.at[pl.ds(k * rows, rows)],
                dst_ref=peer_ref.at[pl.ds(k * rows, rows)],
                send_sem=send_sems.at[k],
                recv_sem=recv_sems.at[k],
                device_id=peer,
                device_id_type=pl.DeviceIdType.MESH,
            )

        pl.semaphore_signal(
            barrier_sem, inc=1, device_id=peer,
            device_id_type=pl.DeviceIdType.MESH,
        )

        for k in range(_NCHUNK):
            s = jnp.sum(
                x_ref[pl.ds(k * bm, bm), :].astype(jnp.float32), axis=1
            )
            partial_ref[pl.ds(k * rows, rows), :] = s.reshape(rows, 128)
            if k == 0:
                pl.semaphore_wait(barrier_sem, 1)
            chunk_rdma(k).start()

        for k in range(_NCHUNK):
            rdma = chunk_rdma(k)
            rdma.wait_send()
            rdma.wait_recv()
        out_ref[:, :] = partial_ref[:, :] + peer_ref[:, :]

    out = pl.pallas_call(
        body,
        out_shape=jax.ShapeDtypeStruct((m_per // 128, 128), jnp.float32),
        in_specs=[pl.BlockSpec(memory_space=pltpu.VMEM)],
        out_specs=pl.BlockSpec(memory_space=pltpu.VMEM),
        scratch_shapes=[
            pltpu.VMEM((m_per // 128, 128), jnp.float32),
            pltpu.VMEM((m_per // 128, 128), jnp.float32),
            pltpu.SemaphoreType.DMA((_NCHUNK,)),
            pltpu.SemaphoreType.DMA((_NCHUNK,)),
        ],
        compiler_params=pltpu.CompilerParams(collective_id=0),
    )(x)
    return out.reshape(m_per, 1)
